# Initial kernel scaffold; baseline (speedup 1.0000x reference)
import jax
import jax.numpy as jnp
from jax.experimental import pallas as pl


def kernel(inputs, routing_inputs):
    raise NotImplementedError("write your pallas kernel here")


# speedup vs baseline: 6.6919x; 6.6919x over previous
import functools

import jax
import jax.numpy as jnp
from jax import lax
from jax.experimental import pallas as pl
from jax.experimental.pallas import tpu as pltpu
from jax.experimental.pallas import tpu_sc as plsc

ROUTES = 8
B = 32
HW = 28 * 28
RW = 768 // ROUTES
LANES = 16


def _body(in_hbm, rt_hbm, out_hbm, rt_v, rows_v, sem):
    c = lax.axis_index("c")
    s = lax.axis_index("s")
    w = s * 2 + c

    pltpu.sync_copy(rt_hbm.at[w], rt_v)
    v = rt_v[...]
    best = v[0]
    r = jnp.int32(0)
    for i in range(1, ROUTES):
        vi = v[i]
        take = vi > best
        best = jnp.where(take, vi, best)
        r = jnp.where(take, jnp.int32(i), r)

    pltpu.async_copy(in_hbm.at[w, :, r, :], rows_v, sem).wait()
    pltpu.sync_copy(rows_v, out_hbm.at[w])


def kernel(inputs, routing_inputs):
    in4 = inputs.reshape(B, HW, ROUTES, RW)
    rt_pad = jnp.pad(routing_inputs, ((0, 0), (0, LANES - ROUTES)),
                     constant_values=-jnp.inf)

    mesh = plsc.VectorSubcoreMesh(core_axis_name="c", subcore_axis_name="s")
    k = functools.partial(
        pl.kernel,
        out_type=jax.ShapeDtypeStruct((B, HW, RW), jnp.float32),
        mesh=mesh,
        scratch_types=[
            pltpu.VMEM((LANES,), jnp.float32),
            pltpu.VMEM((HW, RW), jnp.float32),
            pltpu.SemaphoreType.DMA,
        ],
    )(_body)
    out = k(in4, rt_pad)
    return out.reshape(B, 28, 28, RW)


# speedup vs baseline: 10.6324x; 1.5889x over previous
import functools

import jax
import jax.numpy as jnp
from jax import lax
from jax.experimental import pallas as pl
from jax.experimental.pallas import tpu as pltpu
from jax.experimental.pallas import tpu_sc as plsc

ROUTES = 8
B = 32
H = 28
W = 28
RW = 768 // ROUTES
LANES = 16
HCH = 4
NCH = H // HCH
WIN = 2 * 128


def _body(in_hbm, rt_hbm, out_hbm, rt_v, a0, a1, b0, b1, sa0, sa1, sb0, sb1):
    c = lax.axis_index("c")
    s = lax.axis_index("s")
    w = s * 2 + c

    pltpu.sync_copy(rt_hbm.at[w], rt_v)
    v = rt_v[...]
    best = v[0]
    r = jnp.int32(0)
    for i in range(1, ROUTES):
        vi = v[i]
        take = vi > best
        best = jnp.where(take, vi, best)
        r = jnp.where(take, jnp.int32(i), r)

    t0 = jnp.minimum(jnp.int32(3) * r // jnp.int32(4), jnp.int32(4))
    win0 = pl.multiple_of(t0 * 128, 128)
    choff = pl.multiple_of(r * RW - win0, 16)

    avmem = (a0, a1)
    bvmem = (b0, b1)
    sa = (sa0, sa1)
    sb = (sb0, sb1)

    def shift(src, dst):
        def bi(i, _):
            def bj(j, _):
                for g in range(RW // LANES):
                    dst[i, j, pl.ds(g * LANES, LANES)] = (
                        src[i, j, pl.ds(choff + g * LANES, LANES)])
                return 0
            return lax.fori_loop(0, W, bj, 0)
        lax.fori_loop(0, HCH, bi, 0)

    pltpu.async_copy(
        in_hbm.at[w, pl.ds(0, HCH), :, pl.ds(win0, WIN)], avmem[0], sa[0])
    for i in range(NCH):
        if i + 1 < NCH:
            pltpu.async_copy(
                in_hbm.at[w, pl.ds((i + 1) * HCH, HCH), :, pl.ds(win0, WIN)],
                avmem[(i + 1) % 2], sa[(i + 1) % 2])
        pltpu.make_async_copy(
            in_hbm.at[w, pl.ds(i * HCH, HCH), :, pl.ds(win0, WIN)],
            avmem[i % 2], sa[i % 2]).wait()
        if i >= 2:
            pltpu.make_async_copy(
                bvmem[i % 2], out_hbm.at[w, pl.ds((i - 2) * HCH, HCH)],
                sb[i % 2]).wait()
        shift(avmem[i % 2], bvmem[i % 2])
        pltpu.async_copy(
            bvmem[i % 2], out_hbm.at[w, pl.ds(i * HCH, HCH)], sb[i % 2])
    for i in (NCH - 2, NCH - 1):
        pltpu.make_async_copy(
            bvmem[i % 2], out_hbm.at[w, pl.ds(i * HCH, HCH)], sb[i % 2]).wait()


def kernel(inputs, routing_inputs):
    rt_pad = jnp.pad(routing_inputs, ((0, 0), (0, LANES - ROUTES)))

    mesh = plsc.VectorSubcoreMesh(core_axis_name="c", subcore_axis_name="s")
    k = functools.partial(
        pl.kernel,
        out_type=jax.ShapeDtypeStruct((B, H, W, RW), jnp.float32),
        mesh=mesh,
        scratch_types=[
            pltpu.VMEM((LANES,), jnp.float32),
            pltpu.VMEM((HCH, W, WIN), jnp.float32),
            pltpu.VMEM((HCH, W, WIN), jnp.float32),
            pltpu.VMEM((HCH, W, RW), jnp.float32),
            pltpu.VMEM((HCH, W, RW), jnp.float32),
            pltpu.SemaphoreType.DMA,
            pltpu.SemaphoreType.DMA,
            pltpu.SemaphoreType.DMA,
            pltpu.SemaphoreType.DMA,
        ],
    )(_body)
    return k(inputs, rt_pad)
